# baseline (speedup 1.0000x reference)
import jax
import jax.numpy as jnp
from jax.experimental import pallas as pl
from jax.experimental.pallas import tpu as pltpu

_B2 = 1000


def _heads_kernel(x_ref, ws_ref, bs_ref, wd_ref, bd_ref, s_ref, d_ref):
    x = x_ref[...].astype(jnp.bfloat16)
    s_ref[...] = (
        jnp.dot(x, ws_ref[...], preferred_element_type=jnp.float32) + bs_ref[...]
    )
    d_ref[...] = (
        jnp.dot(x, wd_ref[...], preferred_element_type=jnp.float32) + bd_ref[...]
    )


def kernel(x, W_cls, b_cls, W_bbox, b_bbox):
    if x.ndim > 2:
        x = x.reshape(x.shape[0], -1)
    N, D = x.shape
    Kc = W_cls.shape[1]
    Kb = W_bbox.shape[1]
    half = N // 2

    x2 = x.reshape(half, 2 * D)

    wdt = W_cls.dtype
    z_c = jnp.zeros((D, Kc), wdt)
    z_b = jnp.zeros((D, Kb), wdt)
    W_s = jnp.block([[W_cls, z_c], [z_c, W_cls]]).astype(jnp.bfloat16)
    W_d = jnp.block([[W_bbox, z_b], [z_b, W_bbox]]).astype(jnp.bfloat16)
    b_s = jnp.concatenate([b_cls, b_cls]).reshape(1, 2 * Kc)
    b_d = jnp.concatenate([b_bbox, b_bbox]).reshape(1, 2 * Kb)

    grid = (half // _B2,)
    s2, d2 = pl.pallas_call(
        _heads_kernel,
        grid=grid,
        in_specs=[
            pl.BlockSpec((_B2, 2 * D), lambda i: (i, 0)),
            pl.BlockSpec((2 * D, 2 * Kc), lambda i: (0, 0)),
            pl.BlockSpec((1, 2 * Kc), lambda i: (0, 0)),
            pl.BlockSpec((2 * D, 2 * Kb), lambda i: (0, 0)),
            pl.BlockSpec((1, 2 * Kb), lambda i: (0, 0)),
        ],
        out_specs=[
            pl.BlockSpec((_B2, 2 * Kc), lambda i: (i, 0)),
            pl.BlockSpec((_B2, 2 * Kb), lambda i: (i, 0)),
        ],
        out_shape=[
            jax.ShapeDtypeStruct((half, 2 * Kc), jnp.float32),
            jax.ShapeDtypeStruct((half, 2 * Kb), jnp.float32),
        ],
        compiler_params=pltpu.CompilerParams(
            dimension_semantics=("parallel",),
        ),
    )(x2, W_s, b_s, W_d, b_d)
    return (s2.reshape(N, Kc), d2.reshape(N, Kb))


# speedup vs baseline: 3.3667x; 3.3667x over previous
import jax
import jax.numpy as jnp
from jax.experimental import pallas as pl
from jax.experimental.pallas import tpu as pltpu

_B = 1000


def _heads_kernel(x_ref, wc_ref, bc_ref, wb_ref, bb_ref, s_hbm, d_hbm,
                  s_buf, d_buf, s_sem, d_sem):
    i = pl.program_id(0)
    nb = pl.num_programs(0)
    buf = jax.lax.rem(i, 2)

    def s_copy(step, b):
        return pltpu.make_async_copy(
            s_buf.at[b], s_hbm.at[pl.ds(step * _B, _B), :], s_sem.at[b]
        )

    def d_copy(step, b):
        return pltpu.make_async_copy(
            d_buf.at[b], d_hbm.at[pl.ds(step * _B, _B), :], d_sem.at[b]
        )

    @pl.when(i >= 2)
    def _():
        s_copy(i - 2, buf).wait()
        d_copy(i - 2, buf).wait()

    x = x_ref[...].astype(jnp.bfloat16)
    s_buf[buf] = (
        jnp.dot(x, wc_ref[...], preferred_element_type=jnp.float32) + bc_ref[...]
    )
    d_buf[buf] = (
        jnp.dot(x, wb_ref[...], preferred_element_type=jnp.float32) + bb_ref[...]
    )

    s_copy(i, buf).start()
    d_copy(i, buf).start()

    @pl.when(i == nb - 1)
    def _():
        s_copy(i - 1, 1 - buf).wait()
        d_copy(i - 1, 1 - buf).wait()
        s_copy(i, buf).wait()
        d_copy(i, buf).wait()


def kernel(x, W_cls, b_cls, W_bbox, b_bbox):
    if x.ndim > 2:
        x = x.reshape(x.shape[0], -1)
    N, D = x.shape
    Kc = W_cls.shape[1]
    Kb = W_bbox.shape[1]
    bc2 = b_cls.reshape(1, Kc)
    bb2 = b_bbox.reshape(1, Kb)
    Wc16 = W_cls.astype(jnp.bfloat16)
    Wb16 = W_bbox.astype(jnp.bfloat16)
    grid = (N // _B,)
    scores, deltas = pl.pallas_call(
        _heads_kernel,
        grid=grid,
        in_specs=[
            pl.BlockSpec((_B, D), lambda i: (i, 0)),
            pl.BlockSpec((D, Kc), lambda i: (0, 0)),
            pl.BlockSpec((1, Kc), lambda i: (0, 0)),
            pl.BlockSpec((D, Kb), lambda i: (0, 0)),
            pl.BlockSpec((1, Kb), lambda i: (0, 0)),
        ],
        out_specs=[
            pl.BlockSpec(memory_space=pl.ANY),
            pl.BlockSpec(memory_space=pl.ANY),
        ],
        out_shape=[
            jax.ShapeDtypeStruct((N, Kc), jnp.float32),
            jax.ShapeDtypeStruct((N, Kb), jnp.float32),
        ],
        scratch_shapes=[
            pltpu.VMEM((2, _B, Kc), jnp.float32),
            pltpu.VMEM((2, _B, Kb), jnp.float32),
            pltpu.SemaphoreType.DMA((2,)),
            pltpu.SemaphoreType.DMA((2,)),
        ],
    )(x, Wc16, bc2, Wb16, bb2)
    return (scores, deltas)


# speedup vs baseline: 3.3874x; 1.0061x over previous
import jax
import jax.numpy as jnp
from jax.experimental import pallas as pl
from jax.experimental.pallas import tpu as pltpu

_B = 1000


def _heads_kernel(x_ref, wc_ref, bc_ref, wb_ref, bb_ref, s_ref, d_ref):
    x = x_ref[...].astype(jnp.bfloat16)
    s_ref[...] = (
        jnp.dot(x, wc_ref[...], preferred_element_type=jnp.float32) + bc_ref[...]
    )
    d_ref[...] = (
        jnp.dot(x, wb_ref[...], preferred_element_type=jnp.float32) + bb_ref[...]
    )


def kernel(x, W_cls, b_cls, W_bbox, b_bbox):
    if x.ndim > 2:
        x = x.reshape(x.shape[0], -1)
    N, D = x.shape
    Kc = W_cls.shape[1]
    Kb = W_bbox.shape[1]
    bc2 = b_cls.reshape(1, Kc)
    bb2 = b_bbox.reshape(1, Kb)
    Wc16 = W_cls.astype(jnp.bfloat16)
    Wb16 = W_bbox.astype(jnp.bfloat16)
    grid = (N // _B,)
    scores, deltas = pl.pallas_call(
        _heads_kernel,
        grid=grid,
        in_specs=[
            pl.BlockSpec((_B, D), lambda i: (i, 0)),
            pl.BlockSpec((D, Kc), lambda i: (0, 0)),
            pl.BlockSpec((1, Kc), lambda i: (0, 0)),
            pl.BlockSpec((D, Kb), lambda i: (0, 0)),
            pl.BlockSpec((1, Kb), lambda i: (0, 0)),
        ],
        out_specs=[
            pl.BlockSpec((_B, Kc), lambda i: (i, 0)),
            pl.BlockSpec((_B, Kb), lambda i: (i, 0)),
        ],
        out_shape=[
            jax.ShapeDtypeStruct((N, Kc), jnp.float32),
            jax.ShapeDtypeStruct((N, Kb), jnp.float32),
        ],
        compiler_params=pltpu.CompilerParams(
            dimension_semantics=("parallel",),
        ),
    )(x, Wc16, bc2, Wb16, bb2)
    return (scores, deltas)
